# Initial kernel scaffold; baseline (speedup 1.0000x reference)
import jax
import jax.numpy as jnp
from jax.experimental import pallas as pl


def kernel(indices, W):
    raise NotImplementedError("write your pallas kernel here")


# speedup vs baseline: 6.3778x; 6.3778x over previous
import jax
import jax.numpy as jnp
from jax import lax
from jax.experimental import pallas as pl
from jax.experimental.pallas import tpu as pltpu
from jax.experimental.pallas import tpu_sc as plsc

VOCAB = 10
NC = 2
NS = 16
NW = NC * NS
LANES = 16
CHUNK = 800
UNROLL = 5


def _gram_body(w_ref, g_ref):
    W = w_ref[:, :]
    i = lax.broadcasted_iota(jnp.int32, (VOCAB, VOCAB * VOCAB), 0)
    p = lax.broadcasted_iota(jnp.int32, (VOCAB, VOCAB * VOCAB), 1)
    ohl = (p // VOCAB == i).astype(jnp.float32)
    ohr = (p % VOCAB == i).astype(jnp.float32)
    wl = lax.dot_general(W, ohl, (((0,), (0,)), ((), ())),
                         preferred_element_type=jnp.float32)
    wr = lax.dot_general(W, ohr, (((0,), (0,)), ((), ())),
                         preferred_element_type=jnp.float32)
    g_ref[:, :] = jnp.sum(wl * wr, axis=0, keepdims=True)


def _expand_body(n_tok, g_hbm, idx_hbm, out_hbm, g_v, idx_v, out_v, sem):
    wid = lax.axis_index("s") * NC + lax.axis_index("c")
    per_w = n_tok // NW
    n_chunks = per_w // CHUNK
    base = wid * per_w

    pltpu.sync_copy(g_hbm, g_v)
    lane = lax.iota(jnp.int32, LANES)
    zero16 = jnp.zeros((LANES,), jnp.int32)
    jvecs = [jnp.full((LANES,), j, jnp.int32) for j in range(VOCAB)]

    def chunk_body(ci, carry):
        t0 = base + ci * CHUNK
        pltpu.sync_copy(idx_hbm.at[pl.ds(t0, CHUNK)], idx_v)

        def grp(gi, c2):
            for u in range(UNROLL):
                g = gi * UNROLL + u
                ids = idx_v[pl.ds(g * LANES, LANES)]
                ids10 = ids * VOCAB
                tloc = g * LANES + lane
                for j in range(VOCAB):
                    vals = plsc.load_gather(g_v, [zero16, ids10 + jvecs[j]])
                    plsc.store_scatter(out_v, [tloc, jvecs[j]], vals)
            return c2

        lax.fori_loop(0, CHUNK // (LANES * UNROLL), grp, 0)
        pltpu.sync_copy(out_v, out_hbm.at[pl.ds(t0, CHUNK)])
        return carry

    lax.fori_loop(0, n_chunks, chunk_body, 0)


def kernel(indices, W):
    B, L = indices.shape
    n_tok = B * L
    idx_flat = indices.astype(jnp.int32).reshape(n_tok)
    W = W.astype(jnp.float32)

    g = pl.pallas_call(
        _gram_body,
        out_shape=jax.ShapeDtypeStruct((1, VOCAB * VOCAB), jnp.float32),
    )(W)

    mesh = plsc.VectorSubcoreMesh(core_axis_name="c", subcore_axis_name="s")
    run = pl.kernel(
        lambda *a: _expand_body(n_tok, *a),
        out_type=jax.ShapeDtypeStruct((n_tok, VOCAB), jnp.float32),
        mesh=mesh,
        scratch_types=[
            pltpu.VMEM((1, VOCAB * VOCAB), jnp.float32),
            pltpu.VMEM((CHUNK,), jnp.int32),
            pltpu.VMEM((CHUNK, VOCAB), jnp.float32),
            pltpu.SemaphoreType.DMA,
        ],
        compiler_params=pltpu.CompilerParams(needs_layout_passes=False),
    )
    out = run(g, idx_flat)
    return out.reshape(B, L, VOCAB)


# speedup vs baseline: 8.7182x; 1.3670x over previous
import jax
import jax.numpy as jnp
from jax import lax
from jax.experimental import pallas as pl
from jax.experimental.pallas import tpu as pltpu
from jax.experimental.pallas import tpu_sc as plsc

VOCAB = 10
NC = 2
NS = 16
NW = NC * NS
LANES = 16
SUPER_R = 8
SUB_R = 2


def _gram_body(w_ref, g_ref):
    W = w_ref[:, :]
    i = lax.broadcasted_iota(jnp.int32, (VOCAB, VOCAB * VOCAB), 0)
    p = lax.broadcasted_iota(jnp.int32, (VOCAB, VOCAB * VOCAB), 1)
    ohl = (p // VOCAB == i).astype(jnp.float32)
    ohr = (p % VOCAB == i).astype(jnp.float32)
    wl = lax.dot_general(W, ohl, (((0,), (0,)), ((), ())),
                         preferred_element_type=jnp.float32)
    wr = lax.dot_general(W, ohr, (((0,), (0,)), ((), ())),
                         preferred_element_type=jnp.float32)
    g_ref[:, :] = jnp.sum(wl * wr, axis=0, keepdims=True)


def _expand_body(B, L, g_hbm, idx_hbm, out_hbm,
                 g_v, idx_v, out_v, g_sem, i_sems, o_sems):
    sub_tok = SUB_R * L
    wid = lax.axis_index("s") * NC + lax.axis_index("c")
    rows_w = B // NW
    n_super = rows_w // SUPER_R
    subs = SUPER_R // SUB_R
    row0 = wid * rows_w
    tok0 = row0 * L

    pltpu.async_copy(g_hbm, g_v, g_sem).wait()
    lane = lax.iota(jnp.int32, LANES)
    zero16 = jnp.zeros((LANES,), jnp.int32)
    jvecs = [jnp.full((LANES,), j, jnp.int32) for j in range(VOCAB)]
    goffs = [g * LANES for g in range(L // LANES)] + [L - LANES]

    def stage(si, b):
        pltpu.async_copy(idx_hbm.at[pl.ds(row0 + si * SUPER_R, SUPER_R)],
                         idx_v.at[b], i_sems[b])

    def wait_stage(b):
        pltpu.make_async_copy(idx_hbm.at[pl.ds(row0, SUPER_R)],
                              idx_v.at[b], i_sems[b]).wait()

    def out_dma(ci, ob):
        return pltpu.make_async_copy(
            out_v.at[ob], out_hbm.at[pl.ds(tok0 + ci * sub_tok, sub_tok)],
            o_sems[ob])

    def compute_sub(b, sub, ob):
        for rr in range(SUB_R):
            rvec = zero16 + sub * SUB_R + rr
            for goff in goffs:
                ids = plsc.load_gather(idx_v.at[b], [rvec, goff + lane])
                ids10 = ids * VOCAB
                tloc = rr * L + goff + lane
                for j in range(VOCAB):
                    vals = plsc.load_gather(g_v, [zero16, ids10 + jvecs[j]])
                    plsc.store_scatter(out_v.at[ob], [tloc, jvecs[j]], vals)

    stage(0, 0)
    stage(1, 1)

    def super_pair(pi, carry):
        for b in range(2):
            si = pi * 2 + b
            wait_stage(b)

            def sub_pair(spi, c2):
                for ob in range(2):
                    sub = spi * 2 + ob
                    ci = si * subs + sub

                    @pl.when(ci >= 2)
                    def _():
                        out_dma(ci, ob).wait()
                    compute_sub(b, sub, ob)
                    out_dma(ci, ob).start()
                return c2

            lax.fori_loop(0, subs // 2, sub_pair, 0)

            @pl.when(si + 2 < n_super)
            def _():
                stage(si + 2, b)
        return carry

    lax.fori_loop(0, n_super // 2, super_pair, 0)
    out_dma(0, 0).wait()
    out_dma(0, 1).wait()


def kernel(indices, W):
    B, L = indices.shape
    n_tok = B * L
    idx2d = indices.astype(jnp.int32)
    W = W.astype(jnp.float32)

    g = pl.pallas_call(
        _gram_body,
        out_shape=jax.ShapeDtypeStruct((1, VOCAB * VOCAB), jnp.float32),
    )(W)

    mesh = plsc.VectorSubcoreMesh(core_axis_name="c", subcore_axis_name="s")
    run = pl.kernel(
        lambda *a: _expand_body(B, L, *a),
        out_type=jax.ShapeDtypeStruct((n_tok, VOCAB), jnp.float32),
        mesh=mesh,
        scratch_types=[
            pltpu.VMEM((1, VOCAB * VOCAB), jnp.float32),
            pltpu.VMEM((2, SUPER_R, L), jnp.int32),
            pltpu.VMEM((2, SUB_R * L, VOCAB), jnp.float32),
            pltpu.SemaphoreType.DMA,
            [pltpu.SemaphoreType.DMA] * 2,
            [pltpu.SemaphoreType.DMA] * 2,
        ],
        compiler_params=pltpu.CompilerParams(needs_layout_passes=False),
    )
    out = run(g, idx2d)
    return out.reshape(B, L, VOCAB)
